# Initial kernel scaffold; baseline (speedup 1.0000x reference)
import functools

import jax
import jax.numpy as jnp
from jax import lax
from jax.experimental import pallas as pl
from jax.experimental.pallas import tpu as pltpu
from jax.experimental.pallas import tpu_sc as plsc

_F = 26
_B = 4096
_L = 20
_V = 100000
_D = 64

_NW = 32
_FB = _B * _F
_BAGS_W = _FB // _NW
_NB = 32
_ROWS_C = _NB * _L
_GROWS = 128
_NG = _ROWS_C // _GROWS
_NCHUNK = _BAGS_W // _NB
_IDXR_W = _BAGS_W * _L // 128
_NLANE = 16


@functools.partial(
    pl.kernel,
    mesh=plsc.VectorSubcoreMesh(core_axis_name="c", subcore_axis_name="s"),
    out_type=jax.ShapeDtypeStruct((_FB, _D), jnp.float32),
    scratch_types=[
        pltpu.VMEM((2, _NG, 128), jnp.int32),
        pltpu.VMEM((2, _ROWS_C, _D), jnp.float32),
        pltpu.VMEM((_NB, _D), jnp.float32),
        pltpu.SemaphoreType.DMA,
        pltpu.SemaphoreType.DMA,
    ],
)
def _sc_lookup(idx_hbm, tab_hbm, out_hbm, idxv, rowsv, outv, gsem0, gsem1):
    gsem = (gsem0, gsem1)
    wid = lax.axis_index("c") * 16 + lax.axis_index("s")
    g0 = wid * _BAGS_W
    r0w = wid * _IDXR_W

    def fire(cn, s):
        pltpu.sync_copy(idx_hbm.at[pl.ds(r0w + cn * _NG, _NG)], idxv.at[s])
        for j in range(_NG):
            pltpu.async_copy(
                tab_hbm.at[idxv.at[s, j]],
                rowsv.at[s, pl.ds(j * _GROWS, _GROWS)],
                gsem[s],
            )

    def drain(s):
        pltpu.make_async_copy(
            tab_hbm.at[pl.ds(0, _ROWS_C)], rowsv.at[s], gsem[s]
        ).wait()

    def consume(cn, s):
        def bag(j, carry):
            base = j * _L
            for k in range(_D // _NLANE):
                acc = rowsv[s, base, pl.ds(k * _NLANE, _NLANE)]
                for l in range(1, _L):
                    acc = acc + rowsv[s, base + l, pl.ds(k * _NLANE, _NLANE)]
                outv[j, pl.ds(k * _NLANE, _NLANE)] = acc
            return carry

        lax.fori_loop(0, _NB, bag, 0)
        pltpu.sync_copy(outv, out_hbm.at[pl.ds(g0 + cn * _NB, _NB)])

    fire(0, 0)
    fire(1, 1)

    def step(c2, carry):
        for s in range(2):
            cn = 2 * c2 + s
            drain(s)
            consume(cn, s)

            @pl.when(cn + 2 < _NCHUNK)
            def _():
                fire(cn + 2, s)

        return carry

    lax.fori_loop(0, _NCHUNK // 2, step, 0)


def kernel(indices, tables):
    offs = (jnp.arange(_F, dtype=jnp.int32) * _V)[None, :, None]
    idx = indices.astype(jnp.int32).transpose(1, 0, 2) + offs
    idx2 = idx.reshape(_FB * _L // 128, 128)
    tab = tables.reshape(_F * _V, _D)
    out = _sc_lookup(idx2, tab)
    return out.reshape(_B, _F * _D)


# speedup vs baseline: 1.4933x; 1.4933x over previous
import functools

import jax
import jax.numpy as jnp
from jax import lax
from jax.experimental import pallas as pl
from jax.experimental.pallas import tpu as pltpu
from jax.experimental.pallas import tpu_sc as plsc

_F = 26
_B = 4096
_L = 20
_V = 100000
_D = 64

_NW = 32
_FB = _B * _F
_BAGS_W = _FB // _NW
_NB = 32
_ROWS_C = _NB * _L
_GROWS = 128
_NG = _ROWS_C // _GROWS
_NCHUNK = _BAGS_W // _NB
_IDXR_W = _BAGS_W * _L
_NLANE = 16


@functools.partial(
    pl.kernel,
    mesh=plsc.VectorSubcoreMesh(core_axis_name="c", subcore_axis_name="s"),
    out_type=jax.ShapeDtypeStruct((_FB, _D), jnp.float32),
    compiler_params=pltpu.CompilerParams(use_tc_tiling_on_sc=False),
    scratch_types=[
        pltpu.VMEM((2, _ROWS_C), jnp.int32),
        pltpu.VMEM((2, _ROWS_C, _D), jnp.float32),
        pltpu.VMEM((_NB, _D), jnp.float32),
        pltpu.SemaphoreType.DMA,
        pltpu.SemaphoreType.DMA,
    ],
)
def _sc_lookup(idx_hbm, tab_hbm, out_hbm, idxv, rowsv, outv, gsem0, gsem1):
    gsem = (gsem0, gsem1)
    wid = lax.axis_index("c") * 16 + lax.axis_index("s")
    g0 = wid * _BAGS_W
    r0w = wid * _IDXR_W

    def fire(cn, s):
        pltpu.sync_copy(
            idx_hbm.at[pl.ds(r0w + cn * _ROWS_C, _ROWS_C)], idxv.at[s]
        )
        for j in range(_NG):
            pltpu.async_copy(
                tab_hbm.at[idxv.at[s, pl.ds(j * _GROWS, _GROWS)]],
                rowsv.at[s, pl.ds(j * _GROWS, _GROWS)],
                gsem[s],
            )

    def drain(s):
        pltpu.make_async_copy(
            tab_hbm.at[pl.ds(0, _ROWS_C)], rowsv.at[s], gsem[s]
        ).wait()

    def consume(cn, s):
        def bag(j, carry):
            base = j * _L
            for k in range(_D // _NLANE):
                acc = rowsv[s, base, pl.ds(k * _NLANE, _NLANE)]
                for l in range(1, _L):
                    acc = acc + rowsv[s, base + l, pl.ds(k * _NLANE, _NLANE)]
                outv[j, pl.ds(k * _NLANE, _NLANE)] = acc
            return carry

        lax.fori_loop(0, _NB, bag, 0)
        pltpu.sync_copy(outv, out_hbm.at[pl.ds(g0 + cn * _NB, _NB)])

    fire(0, 0)
    fire(1, 1)

    def step(c2, carry):
        for s in range(2):
            cn = 2 * c2 + s
            drain(s)
            consume(cn, s)

            @pl.when(cn + 2 < _NCHUNK)
            def _():
                fire(cn + 2, s)

        return carry

    lax.fori_loop(0, _NCHUNK // 2, step, 0)


def kernel(indices, tables):
    offs = (jnp.arange(_F, dtype=jnp.int32) * _V)[None, :, None]
    idx = indices.astype(jnp.int32).transpose(1, 0, 2) + offs
    idx2 = idx.reshape(_FB * _L)
    tab = tables.reshape(_F * _V, _D)
    out = _sc_lookup(idx2, tab)
    return out.reshape(_B, _F * _D)


# speedup vs baseline: 1.4988x; 1.0037x over previous
import functools

import jax
import jax.numpy as jnp
from jax import lax
from jax.experimental import pallas as pl
from jax.experimental.pallas import tpu as pltpu
from jax.experimental.pallas import tpu_sc as plsc

_F = 26
_B = 4096
_L = 20
_V = 100000
_D = 64

_NW = 32
_FB = _B * _F
_BAGS_W = _FB // _NW
_NB = 32
_ROWS_C = _NB * _L
_GROWS = 128
_NG = _ROWS_C // _GROWS
_NCHUNK = _BAGS_W // _NB
_NLANE = 16


@functools.partial(
    pl.kernel,
    mesh=plsc.VectorSubcoreMesh(core_axis_name="c", subcore_axis_name="s"),
    out_type=jax.ShapeDtypeStruct((_FB, _D), jnp.float32),
    compiler_params=pltpu.CompilerParams(use_tc_tiling_on_sc=False),
    scratch_types=[
        pltpu.VMEM((2, _ROWS_C), jnp.int32),
        pltpu.VMEM((2, _ROWS_C, _D), jnp.float32),
        pltpu.VMEM((_NB, _D), jnp.float32),
        pltpu.VMEM((_NB,), jnp.int32),
        pltpu.SemaphoreType.DMA,
        pltpu.SemaphoreType.DMA,
    ],
)
def _sc_lookup(idx_hbm, tab_hbm, out_hbm, idxv, rowsv, outv, oidx, gsem0, gsem1):
    gsem = (gsem0, gsem1)
    wid = lax.axis_index("c") * 16 + lax.axis_index("s")
    g0 = wid * _BAGS_W

    def fire(cn, s):
        base_g = g0 + cn * _NB
        f_s = base_g >> 12
        pltpu.sync_copy(idx_hbm.at[pl.ds(base_g * _L, _ROWS_C)], idxv.at[s])
        for j in range(_NG):
            pltpu.async_copy(
                tab_hbm.at[f_s].at[idxv.at[s, pl.ds(j * _GROWS, _GROWS)]],
                rowsv.at[s, pl.ds(j * _GROWS, _GROWS)],
                gsem[s],
            )

    def drain(s):
        pltpu.make_async_copy(
            tab_hbm.at[0].at[pl.ds(0, _ROWS_C)], rowsv.at[s], gsem[s]
        ).wait()

    def consume(cn, s):
        def bag(j, carry):
            base = j * _L
            for k in range(_D // _NLANE):
                acc = rowsv[s, base, pl.ds(k * _NLANE, _NLANE)]
                for l in range(1, _L):
                    acc = acc + rowsv[s, base + l, pl.ds(k * _NLANE, _NLANE)]
                outv[j, pl.ds(k * _NLANE, _NLANE)] = acc
            return carry

        lax.fori_loop(0, _NB, bag, 0)

        base_g = g0 + cn * _NB
        f_s = base_g >> 12
        b0 = base_g & (_B - 1)
        i16 = lax.iota(jnp.int32, _NLANE)
        oidx[pl.ds(0, _NLANE)] = (b0 + i16) * _F + f_s
        oidx[pl.ds(_NLANE, _NLANE)] = (b0 + _NLANE + i16) * _F + f_s
        pltpu.sync_copy(outv, out_hbm.at[oidx])

    fire(0, 0)
    fire(1, 1)

    def step(c2, carry):
        for s in range(2):
            cn = 2 * c2 + s
            drain(s)
            consume(cn, s)

            @pl.when(cn + 2 < _NCHUNK)
            def _():
                fire(cn + 2, s)

        return carry

    lax.fori_loop(0, _NCHUNK // 2, step, 0)


def kernel(indices, tables):
    idx_flat = indices.astype(jnp.int32).reshape(_F * _B * _L)
    out = _sc_lookup(idx_flat, tables)
    return out.reshape(_B, _F * _D)
